# Initial kernel scaffold; baseline (speedup 1.0000x reference)
import functools

import jax
import jax.numpy as jnp
from jax import lax
from jax.experimental import pallas as pl
from jax.experimental.pallas import tpu as pltpu
from jax.experimental.pallas import tpu_sc as plsc

BS, FL, BAG, D = 1024, 100, 20, 32
NB = FL * BAG
RB = 128
NC, NS = 2, 16
NW = NC * NS
BPW = BS // NW
MAXN = 1.0
EPS = 1e-7


def _sc_hist(idx_flat, wts_flat):
    mesh = plsc.VectorSubcoreMesh(core_axis_name="c", subcore_axis_name="s")

    @functools.partial(
        pl.kernel,
        out_type=jax.ShapeDtypeStruct((BS * RB,), jnp.float32),
        mesh=mesh,
        scratch_types=[
            pltpu.VMEM((NB,), jnp.int32),
            pltpu.VMEM((NB,), jnp.float32),
            pltpu.VMEM((BPW * RB,), jnp.float32),
            pltpu.SemaphoreType.DMA,
        ],
    )
    def k(idx_hbm, w_hbm, out_hbm, idx_v, w_v, acc_v, sem):
        wid = lax.axis_index("s") * NC + lax.axis_index("c")
        base = wid * BPW

        zero = jnp.zeros((16,), jnp.float32)

        @pl.loop(0, BPW * RB, step=16)
        def _(i):
            acc_v[pl.ds(i, 16)] = zero

        @pl.loop(0, BPW)
        def _(b):
            pltpu.sync_copy(idx_hbm.at[base + b], idx_v)
            pltpu.sync_copy(w_hbm.at[base + b], w_v)
            off = b * RB

            @pl.loop(0, NB, step=16)
            def _(c):
                ic = idx_v[pl.ds(c, 16)] + off
                wc = w_v[pl.ds(c, 16)]
                plsc.addupdate_scatter(acc_v, [ic], wc)

        pltpu.sync_copy(acc_v, out_hbm.at[pl.ds(base * RB, BPW * RB)])

    return k(idx_flat, wts_flat)


def _tc_combine(W, mask_pad, pt_pad, vt_pad):
    def body(w_ref, m_ref, pt_ref, vt_ref, o_ref):
        def renorm(x):
            n = jnp.sqrt(jnp.sum(x * x, axis=-1, keepdims=True))
            return x * jnp.minimum(1.0, MAXN / jnp.maximum(n, EPS))

        vt = renorm(vt_ref[...])
        pt = renorm(pt_ref[...])
        acc = jnp.dot(w_ref[...], vt, preferred_element_type=jnp.float32,
                      precision=lax.Precision.HIGHEST)
        acc = acc + jnp.dot(m_ref[...], pt, preferred_element_type=jnp.float32,
                            precision=lax.Precision.HIGHEST)
        o_ref[...] = acc * (1.0 / FL)

    return pl.pallas_call(
        body,
        out_shape=jax.ShapeDtypeStruct((BS, D), jnp.float32),
    )(W, mask_pad, pt_pad, vt_pad)


def kernel(value_bin_ind, bin_mask, pos_table, val_table):
    idx_flat = value_bin_ind.reshape(BS, NB).astype(jnp.int32)
    wts_flat = jnp.repeat(bin_mask.astype(jnp.float32), BAG, axis=1)
    W = _sc_hist(idx_flat, wts_flat).reshape(BS, RB)
    mask_pad = jnp.pad(bin_mask.astype(jnp.float32), ((0, 0), (0, RB - FL)))
    pt_pad = jnp.pad(pos_table, ((0, RB - FL), (0, 0)))
    vt_pad = jnp.pad(val_table, ((0, RB - FL), (0, 0)))
    return _tc_combine(W, mask_pad, pt_pad, vt_pad)


# speedup vs baseline: 103.2578x; 103.2578x over previous
import dataclasses
import functools

import jax
import jax.numpy as jnp
from jax import lax
from jax.experimental import pallas as pl
from jax.experimental.pallas import tpu as pltpu
from jax.experimental.pallas import tpu_sc as plsc

BS, FL, BAG, D = 1024, 100, 20, 32
NB = FL * BAG
RB = 128
NC, NS = 2, 16
NW = NC * NS
BPW = BS // NW
MAXN = 1.0
EPS = 1e-7


def _sc_hist(idx_flat, wts_flat):
    mesh = plsc.VectorSubcoreMesh(core_axis_name="c", subcore_axis_name="s")
    cp = pltpu.CompilerParams()
    if "needs_layout_passes" in pltpu.CompilerParams.__dataclass_fields__:
        cp = dataclasses.replace(cp, needs_layout_passes=False)

    @functools.partial(
        pl.kernel,
        compiler_params=cp,
        out_type=jax.ShapeDtypeStruct((BS * RB,), jnp.float32),
        mesh=mesh,
        scratch_types=[
            pltpu.VMEM((NB,), jnp.int32),
            pltpu.VMEM((NB,), jnp.float32),
            pltpu.VMEM((BPW * RB,), jnp.float32),
            pltpu.SemaphoreType.DMA,
        ],
    )
    def k(idx_hbm, w_hbm, out_hbm, idx_v, w_v, acc_v, sem):
        wid = lax.axis_index("s") * NC + lax.axis_index("c")
        base = wid * BPW

        zero = jnp.zeros((16,), jnp.float32)

        @pl.loop(0, BPW * RB, step=16)
        def _(i):
            acc_v[pl.ds(i, 16)] = zero

        @pl.loop(0, BPW)
        def _(b):
            pltpu.sync_copy(idx_hbm.at[base + b], idx_v)
            pltpu.sync_copy(w_hbm.at[base + b], w_v)
            off = b * RB

            @pl.loop(0, NB, step=16)
            def _(c):
                ic = idx_v[pl.ds(c, 16)] + off
                wc = w_v[pl.ds(c, 16)]
                plsc.addupdate_scatter(acc_v, [ic], wc)

        pltpu.sync_copy(acc_v, out_hbm.at[pl.ds(base * RB, BPW * RB)])

    return k(idx_flat, wts_flat)


def _tc_combine(W, mask_pad, pt_pad, vt_pad):
    def body(w_ref, m_ref, pt_ref, vt_ref, o_ref):
        def renorm(x):
            n = jnp.sqrt(jnp.sum(x * x, axis=-1, keepdims=True))
            return x * jnp.minimum(1.0, MAXN / jnp.maximum(n, EPS))

        vt = renorm(vt_ref[...])
        pt = renorm(pt_ref[...])
        acc = jnp.dot(w_ref[...], vt, preferred_element_type=jnp.float32,
                      precision=lax.Precision.HIGHEST)
        acc = acc + jnp.dot(m_ref[...], pt, preferred_element_type=jnp.float32,
                            precision=lax.Precision.HIGHEST)
        o_ref[...] = acc * (1.0 / FL)

    return pl.pallas_call(
        body,
        out_shape=jax.ShapeDtypeStruct((BS, D), jnp.float32),
    )(W, mask_pad, pt_pad, vt_pad)


def kernel(value_bin_ind, bin_mask, pos_table, val_table):
    idx_flat = value_bin_ind.reshape(BS, NB).astype(jnp.int32)
    wts_flat = jnp.repeat(bin_mask.astype(jnp.float32), BAG, axis=1)
    W = _sc_hist(idx_flat, wts_flat).reshape(BS, RB)
    mask_pad = jnp.pad(bin_mask.astype(jnp.float32), ((0, 0), (0, RB - FL)))
    pt_pad = jnp.pad(pos_table, ((0, RB - FL), (0, 0)))
    vt_pad = jnp.pad(val_table, ((0, RB - FL), (0, 0)))
    return _tc_combine(W, mask_pad, pt_pad, vt_pad)
